# Initial kernel scaffold; baseline (speedup 1.0000x reference)
import jax
import jax.numpy as jnp
from jax.experimental import pallas as pl


def kernel(sequence, table):
    raise NotImplementedError("write your pallas kernel here")


# speedup vs baseline: 9.1250x; 9.1250x over previous
import functools

import jax
import jax.numpy as jnp
from jax import lax
from jax.experimental import pallas as pl
from jax.experimental.pallas import tpu as pltpu
from jax.experimental.pallas import tpu_sc as plsc

_VOCAB = 100000
_EMBDIM = 128
_BATCH = 4096
_SEQLEN = 200

_B = _BATCH * _SEQLEN
_NC = 2
_NS = 16
_NW = _NC * _NS
_B_PER_W = _B // _NW
_CHUNK = 128
_NCHUNKS = _B_PER_W // _CHUNK
_NBUF = 4


def _make_gather():
    mesh = plsc.VectorSubcoreMesh(core_axis_name="c", subcore_axis_name="s")

    @functools.partial(
        pl.kernel,
        mesh=mesh,
        out_type=jax.ShapeDtypeStruct((_B, _EMBDIM), jnp.float32),
        scratch_types=[
            pltpu.VMEM((_NCHUNKS, _CHUNK), jnp.int32),
            pltpu.VMEM((_NBUF, _CHUNK, _EMBDIM), jnp.float32),
        ]
        + [pltpu.SemaphoreType.DMA] * (2 * _NBUF),
    )
    def gather_kernel(idx_hbm, table_hbm, out_hbm, idx_v, rows_v, *sems):
        gsems = sems[:_NBUF]
        osems = sems[_NBUF:]
        wid = lax.axis_index("s") * _NC + lax.axis_index("c")
        base = wid * _B_PER_W

        pltpu.sync_copy(idx_hbm.at[wid], idx_v)

        def start_gather(j, b):
            pltpu.async_copy(table_hbm.at[idx_v.at[j]], rows_v.at[b], gsems[b])

        def wait_gather(j, b):
            pltpu.make_async_copy(
                table_hbm.at[idx_v.at[j]], rows_v.at[b], gsems[b]
            ).wait()

        def out_slice(j):
            return out_hbm.at[pl.ds(base + j * _CHUNK, _CHUNK)]

        def start_out(j, b):
            pltpu.async_copy(rows_v.at[b], out_slice(j), osems[b])

        def wait_out(j, b):
            pltpu.make_async_copy(rows_v.at[b], out_slice(j), osems[b]).wait()

        for b in range(_NBUF):
            start_gather(b, b)

        def group(g0, _):
            g = g0 * _NBUF
            for b in range(_NBUF):
                wait_gather(g + b, b)
                start_out(g + b, b)
            for b in range(_NBUF):
                wait_out(g + b, b)

                @pl.when(g + b + _NBUF < _NCHUNKS)
                def _():
                    start_gather(g + b + _NBUF, b)

            return 0

        lax.fori_loop(0, _NCHUNKS // _NBUF, group, 0)

    return gather_kernel


_gather = _make_gather()


def kernel(sequence, table):
    idx = sequence.astype(jnp.int32).reshape(_NW, _NCHUNKS, _CHUNK)
    out = _gather(idx, table)
    return out.reshape(_BATCH, _SEQLEN, _EMBDIM)
